# baseline (speedup 1.0000x reference)
import jax
import jax.numpy as jnp
from jax import lax
from jax.experimental import pallas as pl
from jax.experimental.pallas import tpu as pltpu

NDCG_W = 0.4
LISTMLE_W = 0.3
BINARY_W = 0.3
K = 10
MARGIN = 0.1
BIG = 1e30
LN2 = 0.6931471805599453


def _body(p_ref, r_ref, a_ref, l_ref, pt_ref, lt_ref, out_ref, am_ref, acc_ref):
    step = pl.program_id(0)
    nsteps = pl.num_programs(0)

    am_ref[...] = jnp.where(l_ref[...] == 1, a_ref[...], BIG)
    bm = jnp.where(lt_ref[0] == 0, pt_ref[0], -BIG)
    mb = MARGIN + bm

    def body(rr, acc):
        ar = am_ref[pl.ds(rr, 1), :]
        for c in range(8):
            acc = acc + jnp.maximum(mb[:, c:c + 1] - ar, 0.0)
        return acc

    acc = lax.fori_loop(0, 160, body, jnp.zeros((128, 128), jnp.float32))
    part = jnp.sum(acc)

    @pl.when(step == 0)
    def _():
        acc_ref[0, 0] = part

    @pl.when(step > 0)
    def _():
        acc_ref[0, 0] = acc_ref[0, 0] + part

    @pl.when(step == nsteps - 1)
    def _():
        P = p_ref[...]
        R = r_ref[...]
        B, D = P.shape
        idx = lax.broadcasted_iota(jnp.int32, (B, D), 1)

        rank_p = jnp.zeros((B, D), jnp.float32)
        rank_r = jnp.zeros((B, D), jnp.float32)
        s_exp = jnp.zeros((B, D), jnp.float32)
        for j in range(D):
            Pj = P[:, j:j + 1]
            Rj = R[:, j:j + 1]
            beats_p = (Pj > P) | ((Pj == P) & (j < idx))
            beats_r = (Rj > R) | ((Rj == R) & (j < idx))
            rank_p = rank_p + beats_p.astype(jnp.float32)
            rank_r = rank_r + beats_r.astype(jnp.float32)
            s_exp = s_exp + jnp.exp(Pj) * (1.0 - beats_r.astype(jnp.float32))

        w_p = jnp.where(rank_p < K, LN2 / jnp.log(rank_p + 2.0), 0.0)
        w_r = jnp.where(rank_r < K, LN2 / jnp.log(rank_r + 2.0), 0.0)
        dcg = jnp.sum(R * w_p, axis=1, keepdims=True)
        idcg = jnp.sum(R * w_r, axis=1, keepdims=True)
        ndcg_loss = 1.0 - jnp.sum(dcg / (idcg + 1e-8)) / B

        listmle = jnp.sum(jnp.log(s_exp + 1e-8) - P) / B

        lab = l_ref[...]
        pA = a_ref[...]
        pos = (lab == 1).astype(jnp.float32)
        neg = (lab == 0).astype(jnp.float32)
        n_pos = jnp.sum(pos)
        n_neg = jnp.sum(neg)
        n = pA.size
        bce = jnp.sum(jnp.maximum(pA, 0.0) - pA * pos
                      + jnp.log(1.0 + jnp.exp(-jnp.abs(pA)))) / n
        pair_sum = acc_ref[0, 0]
        rank_loss = pair_sum / jnp.maximum(n_pos * n_neg, 1.0)
        bin_loss = jnp.where((n_pos > 0) & (n_neg > 0), rank_loss, bce)

        total = (NDCG_W * ndcg_loss + LISTMLE_W * listmle
                 + BINARY_W * bin_loss)
        out_ref[...] = jnp.reshape(total, (1, 1))


def kernel(predictions, relevance_scores, labels):
    B, D = predictions.shape
    flat_p = predictions.reshape(-1)
    flat_l = labels.reshape(-1)
    A = flat_p.reshape(160, 128)
    L = flat_l.reshape(160, 128)
    PT = A.reshape(20, 8, 128).transpose(0, 2, 1)
    LT = L.reshape(20, 8, 128).transpose(0, 2, 1)

    out = pl.pallas_call(
        _body,
        grid=(20,),
        in_specs=[
            pl.BlockSpec((B, D), lambda r: (0, 0)),
            pl.BlockSpec((B, D), lambda r: (0, 0)),
            pl.BlockSpec((160, 128), lambda r: (0, 0)),
            pl.BlockSpec((160, 128), lambda r: (0, 0)),
            pl.BlockSpec((1, 128, 8), lambda r: (r, 0, 0)),
            pl.BlockSpec((1, 128, 8), lambda r: (r, 0, 0)),
        ],
        out_specs=pl.BlockSpec((1, 1), lambda r: (0, 0)),
        out_shape=jax.ShapeDtypeStruct((1, 1), jnp.float32),
        scratch_shapes=[
            pltpu.VMEM((160, 128), jnp.float32),
            pltpu.SMEM((1, 1), jnp.float32),
        ],
    )(predictions, relevance_scores, A, L, PT, LT)
    return out.reshape(())


# speedup vs baseline: 2.3666x; 2.3666x over previous
import jax
import jax.numpy as jnp
from jax import lax
from jax.experimental import pallas as pl
from jax.experimental.pallas import tpu as pltpu

NDCG_W = 0.4
LISTMLE_W = 0.3
BINARY_W = 0.3
K = 10
MARGIN = 0.1
BIG = 1e30
LN2 = 0.6931471805599453


def _body(p_ref, r_ref, a_ref, l_ref, pt_ref, lt_ref, out_ref, am_ref, acc_ref):
    step = pl.program_id(0)
    nsteps = pl.num_programs(0)

    am_ref[...] = jnp.where(l_ref[...] == 1, a_ref[...], BIG)
    bm = jnp.where(lt_ref[0] == 0, pt_ref[0], -BIG)
    mb = MARGIN + bm

    part = jnp.float32(0.0)
    for c in range(8):
        mbb = jnp.broadcast_to(mb[:, c:c + 1], (128, 128))

        def body(rr, acc, mbb=mbb):
            ar = am_ref[pl.ds(rr, 1), :]
            return acc + jnp.maximum(mbb - ar, 0.0)

        acc = lax.fori_loop(0, 160, body, jnp.zeros((128, 128), jnp.float32))
        part = part + jnp.sum(acc)

    @pl.when(step == 0)
    def _():
        acc_ref[0, 0] = part

    @pl.when(step > 0)
    def _():
        acc_ref[0, 0] = acc_ref[0, 0] + part

    @pl.when(step == nsteps - 1)
    def _():
        P = p_ref[...]
        R = r_ref[...]
        B, D = P.shape
        idx = lax.broadcasted_iota(jnp.int32, (B, D), 1)

        rank_p = jnp.zeros((B, D), jnp.float32)
        rank_r = jnp.zeros((B, D), jnp.float32)
        s_exp = jnp.zeros((B, D), jnp.float32)
        for j in range(D):
            Pj = P[:, j:j + 1]
            Rj = R[:, j:j + 1]
            beats_p = (Pj > P) | ((Pj == P) & (j < idx))
            beats_r = (Rj > R) | ((Rj == R) & (j < idx))
            rank_p = rank_p + beats_p.astype(jnp.float32)
            rank_r = rank_r + beats_r.astype(jnp.float32)
            s_exp = s_exp + jnp.exp(Pj) * (1.0 - beats_r.astype(jnp.float32))

        w_p = jnp.where(rank_p < K, LN2 / jnp.log(rank_p + 2.0), 0.0)
        w_r = jnp.where(rank_r < K, LN2 / jnp.log(rank_r + 2.0), 0.0)
        dcg = jnp.sum(R * w_p, axis=1, keepdims=True)
        idcg = jnp.sum(R * w_r, axis=1, keepdims=True)
        ndcg_loss = 1.0 - jnp.sum(dcg / (idcg + 1e-8)) / B

        listmle = jnp.sum(jnp.log(s_exp + 1e-8) - P) / B

        lab = l_ref[...]
        pA = a_ref[...]
        pos = (lab == 1).astype(jnp.float32)
        neg = (lab == 0).astype(jnp.float32)
        n_pos = jnp.sum(pos)
        n_neg = jnp.sum(neg)
        n = pA.size
        bce = jnp.sum(jnp.maximum(pA, 0.0) - pA * pos
                      + jnp.log(1.0 + jnp.exp(-jnp.abs(pA)))) / n
        pair_sum = acc_ref[0, 0]
        rank_loss = pair_sum / jnp.maximum(n_pos * n_neg, 1.0)
        bin_loss = jnp.where((n_pos > 0) & (n_neg > 0), rank_loss, bce)

        total = (NDCG_W * ndcg_loss + LISTMLE_W * listmle
                 + BINARY_W * bin_loss)
        out_ref[...] = jnp.reshape(total, (1, 1))


def kernel(predictions, relevance_scores, labels):
    B, D = predictions.shape
    flat_p = predictions.reshape(-1)
    flat_l = labels.reshape(-1)
    A = flat_p.reshape(160, 128)
    L = flat_l.reshape(160, 128)
    PT = A.reshape(20, 8, 128).transpose(0, 2, 1)
    LT = L.reshape(20, 8, 128).transpose(0, 2, 1)

    out = pl.pallas_call(
        _body,
        grid=(20,),
        in_specs=[
            pl.BlockSpec((B, D), lambda r: (0, 0)),
            pl.BlockSpec((B, D), lambda r: (0, 0)),
            pl.BlockSpec((160, 128), lambda r: (0, 0)),
            pl.BlockSpec((160, 128), lambda r: (0, 0)),
            pl.BlockSpec((1, 128, 8), lambda r: (r, 0, 0)),
            pl.BlockSpec((1, 128, 8), lambda r: (r, 0, 0)),
        ],
        out_specs=pl.BlockSpec((1, 1), lambda r: (0, 0)),
        out_shape=jax.ShapeDtypeStruct((1, 1), jnp.float32),
        scratch_shapes=[
            pltpu.VMEM((160, 128), jnp.float32),
            pltpu.SMEM((1, 1), jnp.float32),
        ],
    )(predictions, relevance_scores, A, L, PT, LT)
    return out.reshape(())


# speedup vs baseline: 2.9931x; 1.2647x over previous
import jax
import jax.numpy as jnp
from jax import lax
from jax.experimental import pallas as pl
from jax.experimental.pallas import tpu as pltpu

NDCG_W = 0.4
LISTMLE_W = 0.3
BINARY_W = 0.3
K = 10
MARGIN = 0.1
BIG = 1e30
LN2 = 0.6931471805599453


def _body(p_ref, r_ref, a_ref, l_ref, pt_ref, lt_ref, out_ref, am_ref, acc_ref):
    step = pl.program_id(0)
    nsteps = pl.num_programs(0)

    am_ref[...] = jnp.where(l_ref[...] == 1, a_ref[...], BIG)
    bm = jnp.where(lt_ref[0] == 0, pt_ref[0], -BIG)
    mb = MARGIN + bm

    part = jnp.float32(0.0)
    for c in range(8):
        mbb = jnp.broadcast_to(mb[:, c:c + 1], (128, 128))

        def body(rr, acc, mbb=mbb):
            base = rr * 4
            for k in range(4):
                ar = am_ref[pl.ds(base + k, 1), :]
                acc = acc + jnp.maximum(mbb - ar, 0.0)
            return acc

        acc = lax.fori_loop(0, 40, body, jnp.zeros((128, 128), jnp.float32))
        part = part + jnp.sum(acc)

    @pl.when(step == 0)
    def _():
        acc_ref[0, 0] = part

    @pl.when(step > 0)
    def _():
        acc_ref[0, 0] = acc_ref[0, 0] + part

    @pl.when(step == nsteps - 1)
    def _():
        P = p_ref[...]
        R = r_ref[...]
        B, D = P.shape
        idx = lax.broadcasted_iota(jnp.int32, (B, D), 1)

        rank_p = jnp.zeros((B, D), jnp.float32)
        rank_r = jnp.zeros((B, D), jnp.float32)
        s_exp = jnp.zeros((B, D), jnp.float32)
        for j in range(D):
            Pj = P[:, j:j + 1]
            Rj = R[:, j:j + 1]
            beats_p = (Pj > P) | ((Pj == P) & (j < idx))
            beats_r = (Rj > R) | ((Rj == R) & (j < idx))
            rank_p = rank_p + beats_p.astype(jnp.float32)
            rank_r = rank_r + beats_r.astype(jnp.float32)
            s_exp = s_exp + jnp.exp(Pj) * (1.0 - beats_r.astype(jnp.float32))

        w_p = jnp.where(rank_p < K, LN2 / jnp.log(rank_p + 2.0), 0.0)
        w_r = jnp.where(rank_r < K, LN2 / jnp.log(rank_r + 2.0), 0.0)
        dcg = jnp.sum(R * w_p, axis=1, keepdims=True)
        idcg = jnp.sum(R * w_r, axis=1, keepdims=True)
        ndcg_loss = 1.0 - jnp.sum(dcg / (idcg + 1e-8)) / B

        listmle = jnp.sum(jnp.log(s_exp + 1e-8) - P) / B

        lab = l_ref[...]
        pA = a_ref[...]
        pos = (lab == 1).astype(jnp.float32)
        neg = (lab == 0).astype(jnp.float32)
        n_pos = jnp.sum(pos)
        n_neg = jnp.sum(neg)
        n = pA.size
        bce = jnp.sum(jnp.maximum(pA, 0.0) - pA * pos
                      + jnp.log(1.0 + jnp.exp(-jnp.abs(pA)))) / n
        pair_sum = acc_ref[0, 0]
        rank_loss = pair_sum / jnp.maximum(n_pos * n_neg, 1.0)
        bin_loss = jnp.where((n_pos > 0) & (n_neg > 0), rank_loss, bce)

        total = (NDCG_W * ndcg_loss + LISTMLE_W * listmle
                 + BINARY_W * bin_loss)
        out_ref[...] = jnp.reshape(total, (1, 1))


def kernel(predictions, relevance_scores, labels):
    B, D = predictions.shape
    flat_p = predictions.reshape(-1)
    flat_l = labels.reshape(-1)
    A = flat_p.reshape(160, 128)
    L = flat_l.reshape(160, 128)
    PT = A.reshape(20, 8, 128).transpose(0, 2, 1)
    LT = L.reshape(20, 8, 128).transpose(0, 2, 1)

    out = pl.pallas_call(
        _body,
        grid=(20,),
        in_specs=[
            pl.BlockSpec((B, D), lambda r: (0, 0)),
            pl.BlockSpec((B, D), lambda r: (0, 0)),
            pl.BlockSpec((160, 128), lambda r: (0, 0)),
            pl.BlockSpec((160, 128), lambda r: (0, 0)),
            pl.BlockSpec((1, 128, 8), lambda r: (r, 0, 0)),
            pl.BlockSpec((1, 128, 8), lambda r: (r, 0, 0)),
        ],
        out_specs=pl.BlockSpec((1, 1), lambda r: (0, 0)),
        out_shape=jax.ShapeDtypeStruct((1, 1), jnp.float32),
        scratch_shapes=[
            pltpu.VMEM((160, 128), jnp.float32),
            pltpu.SMEM((1, 1), jnp.float32),
        ],
    )(predictions, relevance_scores, A, L, PT, LT)
    return out.reshape(())


# speedup vs baseline: 3.2313x; 1.0796x over previous
import jax
import jax.numpy as jnp
from jax import lax
from jax.experimental import pallas as pl
from jax.experimental.pallas import tpu as pltpu

NDCG_W = 0.4
LISTMLE_W = 0.3
BINARY_W = 0.3
K = 10
MARGIN = 0.1
BIG = 1e30
LN2 = 0.6931471805599453


def _body(p_ref, r_ref, a_ref, l_ref, pt_ref, lt_ref, out_ref, am_ref, acc_ref):
    step = pl.program_id(0)
    nsteps = pl.num_programs(0)

    am_ref[...] = jnp.where(l_ref[...] == 1, a_ref[...], BIG)
    bm = jnp.where(lt_ref[0] == 0, pt_ref[0], -BIG)
    mb = MARGIN + bm

    part = jnp.float32(0.0)
    for c in range(8):
        mbb = jnp.broadcast_to(mb[:, c:c + 1], (128, 128))

        def body(rr, acc, mbb=mbb):
            base = rr * 8
            for k in range(8):
                ar = am_ref[pl.ds(base + k, 1), :]
                acc = acc + jnp.maximum(mbb - ar, 0.0)
            return acc

        acc = lax.fori_loop(0, 20, body, jnp.zeros((128, 128), jnp.float32))
        part = part + jnp.sum(acc)

    @pl.when(step == 0)
    def _():
        acc_ref[0, 0] = part

    @pl.when(step > 0)
    def _():
        acc_ref[0, 0] = acc_ref[0, 0] + part

    @pl.when(step == nsteps - 1)
    def _():
        P = p_ref[...]
        R = r_ref[...]
        B, D = P.shape
        idx = lax.broadcasted_iota(jnp.int32, (B, D), 1)

        rank_p = jnp.zeros((B, D), jnp.float32)
        rank_r = jnp.zeros((B, D), jnp.float32)
        s_exp = jnp.zeros((B, D), jnp.float32)
        for j in range(D):
            Pj = P[:, j:j + 1]
            Rj = R[:, j:j + 1]
            beats_p = (Pj > P) | ((Pj == P) & (j < idx))
            beats_r = (Rj > R) | ((Rj == R) & (j < idx))
            rank_p = rank_p + beats_p.astype(jnp.float32)
            rank_r = rank_r + beats_r.astype(jnp.float32)
            s_exp = s_exp + jnp.exp(Pj) * (1.0 - beats_r.astype(jnp.float32))

        w_p = jnp.where(rank_p < K, LN2 / jnp.log(rank_p + 2.0), 0.0)
        w_r = jnp.where(rank_r < K, LN2 / jnp.log(rank_r + 2.0), 0.0)
        dcg = jnp.sum(R * w_p, axis=1, keepdims=True)
        idcg = jnp.sum(R * w_r, axis=1, keepdims=True)
        ndcg_loss = 1.0 - jnp.sum(dcg / (idcg + 1e-8)) / B

        listmle = jnp.sum(jnp.log(s_exp + 1e-8) - P) / B

        lab = l_ref[...]
        pA = a_ref[...]
        pos = (lab == 1).astype(jnp.float32)
        neg = (lab == 0).astype(jnp.float32)
        n_pos = jnp.sum(pos)
        n_neg = jnp.sum(neg)
        n = pA.size
        bce = jnp.sum(jnp.maximum(pA, 0.0) - pA * pos
                      + jnp.log(1.0 + jnp.exp(-jnp.abs(pA)))) / n
        pair_sum = acc_ref[0, 0]
        rank_loss = pair_sum / jnp.maximum(n_pos * n_neg, 1.0)
        bin_loss = jnp.where((n_pos > 0) & (n_neg > 0), rank_loss, bce)

        total = (NDCG_W * ndcg_loss + LISTMLE_W * listmle
                 + BINARY_W * bin_loss)
        out_ref[...] = jnp.reshape(total, (1, 1))


def kernel(predictions, relevance_scores, labels):
    B, D = predictions.shape
    flat_p = predictions.reshape(-1)
    flat_l = labels.reshape(-1)
    A = flat_p.reshape(160, 128)
    L = flat_l.reshape(160, 128)
    PT = A.reshape(20, 8, 128).transpose(0, 2, 1)
    LT = L.reshape(20, 8, 128).transpose(0, 2, 1)

    out = pl.pallas_call(
        _body,
        grid=(20,),
        in_specs=[
            pl.BlockSpec((B, D), lambda r: (0, 0)),
            pl.BlockSpec((B, D), lambda r: (0, 0)),
            pl.BlockSpec((160, 128), lambda r: (0, 0)),
            pl.BlockSpec((160, 128), lambda r: (0, 0)),
            pl.BlockSpec((1, 128, 8), lambda r: (r, 0, 0)),
            pl.BlockSpec((1, 128, 8), lambda r: (r, 0, 0)),
        ],
        out_specs=pl.BlockSpec((1, 1), lambda r: (0, 0)),
        out_shape=jax.ShapeDtypeStruct((1, 1), jnp.float32),
        scratch_shapes=[
            pltpu.VMEM((160, 128), jnp.float32),
            pltpu.SMEM((1, 1), jnp.float32),
        ],
    )(predictions, relevance_scores, A, L, PT, LT)
    return out.reshape(())


# speedup vs baseline: 3.2986x; 1.0208x over previous
import jax
import jax.numpy as jnp
from jax import lax
from jax.experimental import pallas as pl
from jax.experimental.pallas import tpu as pltpu

NDCG_W = 0.4
LISTMLE_W = 0.3
BINARY_W = 0.3
K = 10
MARGIN = 0.1
BIG = 1e30
LN2 = 0.6931471805599453


def _body(p_ref, r_ref, a_ref, l_ref, pt_ref, lt_ref, out_ref, am_ref, acc_ref):
    step = pl.program_id(0)
    nsteps = pl.num_programs(0)

    am_ref[...] = jnp.where(l_ref[...] == 1, a_ref[...], BIG)
    bm = jnp.where(lt_ref[0] == 0, pt_ref[0], -BIG)
    mb = MARGIN + bm

    acc = jnp.where(step == 0, 0.0, acc_ref[...])
    for c in range(8):
        mbb = jnp.broadcast_to(mb[:, c:c + 1], (128, 128))

        def body(rr, acc, mbb=mbb):
            base = rr * 8
            for k in range(8):
                ar = am_ref[pl.ds(base + k, 1), :]
                acc = acc + jnp.maximum(mbb - ar, 0.0)
            return acc

        acc = lax.fori_loop(0, 20, body, acc)
    acc_ref[...] = acc

    @pl.when(step == nsteps - 1)
    def _():
        P = p_ref[...]
        R = r_ref[...]
        B, D = P.shape
        idx = lax.broadcasted_iota(jnp.int32, (B, D), 1)

        rank_p = jnp.zeros((B, D), jnp.float32)
        rank_r = jnp.zeros((B, D), jnp.float32)
        s_exp = jnp.zeros((B, D), jnp.float32)
        for j in range(D):
            Pj = P[:, j:j + 1]
            Rj = R[:, j:j + 1]
            beats_p = (Pj > P) | ((Pj == P) & (j < idx))
            beats_r = (Rj > R) | ((Rj == R) & (j < idx))
            rank_p = rank_p + beats_p.astype(jnp.float32)
            rank_r = rank_r + beats_r.astype(jnp.float32)
            s_exp = s_exp + jnp.exp(Pj) * (1.0 - beats_r.astype(jnp.float32))

        w_p = jnp.where(rank_p < K, LN2 / jnp.log(rank_p + 2.0), 0.0)
        w_r = jnp.where(rank_r < K, LN2 / jnp.log(rank_r + 2.0), 0.0)
        dcg = jnp.sum(R * w_p, axis=1, keepdims=True)
        idcg = jnp.sum(R * w_r, axis=1, keepdims=True)
        ndcg_loss = 1.0 - jnp.sum(dcg / (idcg + 1e-8)) / B

        listmle = jnp.sum(jnp.log(s_exp + 1e-8) - P) / B

        lab = l_ref[...]
        pA = a_ref[...]
        pos = (lab == 1).astype(jnp.float32)
        neg = (lab == 0).astype(jnp.float32)
        n_pos = jnp.sum(pos)
        n_neg = jnp.sum(neg)
        n = pA.size
        bce = jnp.sum(jnp.maximum(pA, 0.0) - pA * pos
                      + jnp.log(1.0 + jnp.exp(-jnp.abs(pA)))) / n
        pair_sum = jnp.sum(acc)
        rank_loss = pair_sum / jnp.maximum(n_pos * n_neg, 1.0)
        bin_loss = jnp.where((n_pos > 0) & (n_neg > 0), rank_loss, bce)

        total = (NDCG_W * ndcg_loss + LISTMLE_W * listmle
                 + BINARY_W * bin_loss)
        out_ref[...] = jnp.reshape(total, (1, 1))


def kernel(predictions, relevance_scores, labels):
    B, D = predictions.shape
    flat_p = predictions.reshape(-1)
    flat_l = labels.reshape(-1)
    A = flat_p.reshape(160, 128)
    L = flat_l.reshape(160, 128)
    PT = A.reshape(20, 8, 128).transpose(0, 2, 1)
    LT = L.reshape(20, 8, 128).transpose(0, 2, 1)

    out = pl.pallas_call(
        _body,
        grid=(20,),
        in_specs=[
            pl.BlockSpec((B, D), lambda r: (0, 0)),
            pl.BlockSpec((B, D), lambda r: (0, 0)),
            pl.BlockSpec((160, 128), lambda r: (0, 0)),
            pl.BlockSpec((160, 128), lambda r: (0, 0)),
            pl.BlockSpec((1, 128, 8), lambda r: (r, 0, 0)),
            pl.BlockSpec((1, 128, 8), lambda r: (r, 0, 0)),
        ],
        out_specs=pl.BlockSpec((1, 1), lambda r: (0, 0)),
        out_shape=jax.ShapeDtypeStruct((1, 1), jnp.float32),
        scratch_shapes=[
            pltpu.VMEM((160, 128), jnp.float32),
            pltpu.VMEM((128, 128), jnp.float32),
        ],
    )(predictions, relevance_scores, A, L, PT, LT)
    return out.reshape(())


# speedup vs baseline: 4.4007x; 1.3341x over previous
import jax
import jax.numpy as jnp
from jax import lax
from jax.experimental import pallas as pl
from jax.experimental.pallas import tpu as pltpu

NDCG_W = 0.4
LISTMLE_W = 0.3
BINARY_W = 0.3
K = 10
MARGIN = 0.1
BIG = 1e30
LN2 = 0.6931471805599453


def _body(p_ref, r_ref, a_ref, l_ref, pt_ref, lt_ref, out_ref, am_ref, acc_ref):
    step = pl.program_id(0)
    nsteps = pl.num_programs(0)

    am_ref[...] = jnp.where(l_ref[...] == 1, a_ref[...], BIG)
    bm = jnp.where(lt_ref[0] == 0, pt_ref[0], -BIG)
    mb = MARGIN + bm

    acc = jnp.where(step == 0, 0.0, acc_ref[...])
    for c in range(8):
        mbb = jnp.broadcast_to(mb[:, c:c + 1], (128, 128)).astype(jnp.bfloat16)

        def body(rr, acc, mbb=mbb):
            base = rr * 8
            acc8 = jnp.zeros((128, 128), jnp.bfloat16)
            for k in range(8):
                ar = am_ref[pl.ds(base + k, 1), :]
                acc8 = acc8 + jnp.maximum(mbb - ar.astype(jnp.bfloat16), 0)
            return acc + acc8.astype(jnp.float32)

        acc = lax.fori_loop(0, 20, body, acc)
    acc_ref[...] = acc

    @pl.when(step == nsteps - 1)
    def _():
        P = p_ref[...]
        R = r_ref[...]
        B, D = P.shape
        idx = lax.broadcasted_iota(jnp.int32, (B, D), 1)

        rank_p = jnp.zeros((B, D), jnp.float32)
        rank_r = jnp.zeros((B, D), jnp.float32)
        s_exp = jnp.zeros((B, D), jnp.float32)
        for j in range(D):
            Pj = P[:, j:j + 1]
            Rj = R[:, j:j + 1]
            beats_p = (Pj > P) | ((Pj == P) & (j < idx))
            beats_r = (Rj > R) | ((Rj == R) & (j < idx))
            rank_p = rank_p + beats_p.astype(jnp.float32)
            rank_r = rank_r + beats_r.astype(jnp.float32)
            s_exp = s_exp + jnp.exp(Pj) * (1.0 - beats_r.astype(jnp.float32))

        w_p = jnp.where(rank_p < K, LN2 / jnp.log(rank_p + 2.0), 0.0)
        w_r = jnp.where(rank_r < K, LN2 / jnp.log(rank_r + 2.0), 0.0)
        dcg = jnp.sum(R * w_p, axis=1, keepdims=True)
        idcg = jnp.sum(R * w_r, axis=1, keepdims=True)
        ndcg_loss = 1.0 - jnp.sum(dcg / (idcg + 1e-8)) / B

        listmle = jnp.sum(jnp.log(s_exp + 1e-8) - P) / B

        lab = l_ref[...]
        pA = a_ref[...]
        pos = (lab == 1).astype(jnp.float32)
        neg = (lab == 0).astype(jnp.float32)
        n_pos = jnp.sum(pos)
        n_neg = jnp.sum(neg)
        n = pA.size
        bce = jnp.sum(jnp.maximum(pA, 0.0) - pA * pos
                      + jnp.log(1.0 + jnp.exp(-jnp.abs(pA)))) / n
        pair_sum = jnp.sum(acc)
        rank_loss = pair_sum / jnp.maximum(n_pos * n_neg, 1.0)
        bin_loss = jnp.where((n_pos > 0) & (n_neg > 0), rank_loss, bce)

        total = (NDCG_W * ndcg_loss + LISTMLE_W * listmle
                 + BINARY_W * bin_loss)
        out_ref[...] = jnp.reshape(total, (1, 1))


def kernel(predictions, relevance_scores, labels):
    B, D = predictions.shape
    flat_p = predictions.reshape(-1)
    flat_l = labels.reshape(-1)
    A = flat_p.reshape(160, 128)
    L = flat_l.reshape(160, 128)
    PT = A.reshape(20, 8, 128).transpose(0, 2, 1)
    LT = L.reshape(20, 8, 128).transpose(0, 2, 1)

    out = pl.pallas_call(
        _body,
        grid=(20,),
        in_specs=[
            pl.BlockSpec((B, D), lambda r: (0, 0)),
            pl.BlockSpec((B, D), lambda r: (0, 0)),
            pl.BlockSpec((160, 128), lambda r: (0, 0)),
            pl.BlockSpec((160, 128), lambda r: (0, 0)),
            pl.BlockSpec((1, 128, 8), lambda r: (r, 0, 0)),
            pl.BlockSpec((1, 128, 8), lambda r: (r, 0, 0)),
        ],
        out_specs=pl.BlockSpec((1, 1), lambda r: (0, 0)),
        out_shape=jax.ShapeDtypeStruct((1, 1), jnp.float32),
        scratch_shapes=[
            pltpu.VMEM((160, 128), jnp.float32),
            pltpu.VMEM((128, 128), jnp.float32),
        ],
    )(predictions, relevance_scores, A, L, PT, LT)
    return out.reshape(())


# speedup vs baseline: 25.7178x; 5.8441x over previous
import jax
import jax.numpy as jnp
from jax import lax
from jax.experimental import pallas as pl
from jax.experimental.pallas import tpu as pltpu

NDCG_W = 0.4
LISTMLE_W = 0.3
BINARY_W = 0.3
K = 10
MARGIN = 0.1
LN2 = 0.6931471805599453

N_REAL = 20480
NROW = 256
NPAD = NROW * 128
FILLER = 0x7F800001


def _mangle(u):
    m = u >> 31
    return u ^ (m & 0x7FFFFFFF)


def _roll(x, shift, axis):
    return jnp.roll(x, shift, axis=axis)


def _body(p_ref, r_ref, a_ref, l_ref, out_ref):
    A = a_ref[...]
    Lab = l_ref[...]

    merged = jnp.where(Lab == 0, A, A - MARGIN)
    u = lax.bitcast_convert_type(merged, jnp.int32)
    s = _mangle(u)
    keys160 = (s & -2) | jnp.where(Lab == 1, 1, 0)
    x = jnp.concatenate(
        [keys160, jnp.full((NROW - 160, 128), FILLER, jnp.int32)], axis=0)

    iota_l = lax.broadcasted_iota(jnp.int32, (NROW, 128), 1)
    iota_r = lax.broadcasted_iota(jnp.int32, (NROW, 128), 0)

    for stage in range(1, 16):
        k = 1 << stage
        if k < 128:
            up = (iota_l & k) == 0
        elif k < NPAD:
            up = (iota_r & (k >> 7)) == 0
        else:
            up = None
        j = k >> 1
        while j >= 1:
            if j < 128:
                lower = (iota_l & j) == 0
                pm = _roll(x, -j, 1)
                pp = _roll(x, j, 1)
            else:
                jr = j >> 7
                lower = (iota_r & jr) == 0
                pm = _roll(x, -jr, 0)
                pp = _roll(x, jr, 0)
            p = jnp.where(lower, pm, pp)
            want_max = jnp.logical_xor(up, lower) if up is not None else ~lower
            x = jnp.where(want_max, jnp.maximum(x, p), jnp.minimum(x, p))
            j >>= 1

    pos_tag = (x & 1) == 1
    sk = x & -2
    v = lax.bitcast_convert_type(_mangle(sk), jnp.float32)
    flat = iota_r * 128 + iota_l
    real = flat < N_REAL
    negm = (~pos_tag) & real

    cnt = jnp.where(negm, 1.0, 0.0)
    val = jnp.where(negm, v, 0.0)

    for sh in (1, 2, 4, 8, 16, 32, 64):
        lm = iota_l >= sh
        cnt = cnt + jnp.where(lm, _roll(cnt, sh, 1), 0.0)
        val = val + jnp.where(lm, _roll(val, sh, 1), 0.0)
    rt_c = cnt[:, 127:128]
    rt_v = val[:, 127:128]
    ic = rt_c
    iv = rt_v
    iota_rc = iota_r[:, 0:1]
    for sh in (1, 2, 4, 8, 16, 32, 64, 128):
        rm = iota_rc >= sh
        ic = ic + jnp.where(rm, _roll(ic, sh, 0), 0.0)
        iv = iv + jnp.where(rm, _roll(iv, sh, 0), 0.0)
    cnt = cnt + (ic - rt_c)
    val = val + (iv - rt_v)
    n_neg_s = ic[NROW - 1:NROW, :]
    s_tot = iv[NROW - 1:NROW, :]

    contrib = jnp.where(pos_tag & real,
                        (s_tot - val) - v * (n_neg_s - cnt), 0.0)
    pair_sum = jnp.sum(contrib)

    P = p_ref[...]
    R = r_ref[...]
    B, D = P.shape
    idx = lax.broadcasted_iota(jnp.int32, (B, D), 1)

    rank_p = jnp.zeros((B, D), jnp.float32)
    rank_r = jnp.zeros((B, D), jnp.float32)
    s_exp = jnp.zeros((B, D), jnp.float32)
    for j in range(D):
        Pj = P[:, j:j + 1]
        Rj = R[:, j:j + 1]
        beats_p = (Pj > P) | ((Pj == P) & (j < idx))
        beats_r = (Rj > R) | ((Rj == R) & (j < idx))
        rank_p = rank_p + beats_p.astype(jnp.float32)
        rank_r = rank_r + beats_r.astype(jnp.float32)
        s_exp = s_exp + jnp.exp(Pj) * (1.0 - beats_r.astype(jnp.float32))

    w_p = jnp.where(rank_p < K, LN2 / jnp.log(rank_p + 2.0), 0.0)
    w_r = jnp.where(rank_r < K, LN2 / jnp.log(rank_r + 2.0), 0.0)
    dcg = jnp.sum(R * w_p, axis=1, keepdims=True)
    idcg = jnp.sum(R * w_r, axis=1, keepdims=True)
    ndcg_loss = 1.0 - jnp.sum(dcg / (idcg + 1e-8)) / B

    listmle = jnp.sum(jnp.log(s_exp + 1e-8) - P) / B

    pos = (Lab == 1).astype(jnp.float32)
    n_pos = jnp.sum(pos)
    n_neg = jnp.float32(N_REAL) - n_pos
    bce = jnp.sum(jnp.maximum(A, 0.0) - A * pos
                  + jnp.log(1.0 + jnp.exp(-jnp.abs(A)))) / N_REAL
    rank_loss = pair_sum / jnp.maximum(n_pos * n_neg, 1.0)
    bin_loss = jnp.where((n_pos > 0) & (n_neg > 0), rank_loss, bce)

    total = NDCG_W * ndcg_loss + LISTMLE_W * listmle + BINARY_W * bin_loss
    out_ref[...] = jnp.reshape(total, (1, 1))


def kernel(predictions, relevance_scores, labels):
    B, D = predictions.shape
    A = predictions.reshape(160, 128)
    L = labels.reshape(160, 128)

    out = pl.pallas_call(
        _body,
        in_specs=[
            pl.BlockSpec((B, D), lambda: (0, 0)),
            pl.BlockSpec((B, D), lambda: (0, 0)),
            pl.BlockSpec((160, 128), lambda: (0, 0)),
            pl.BlockSpec((160, 128), lambda: (0, 0)),
        ],
        out_specs=pl.BlockSpec((1, 1), lambda: (0, 0)),
        out_shape=jax.ShapeDtypeStruct((1, 1), jnp.float32),
    )(predictions, relevance_scores, A, L)
    return out.reshape(())


# speedup vs baseline: 43.8365x; 1.7045x over previous
import jax
import jax.numpy as jnp
from jax import lax
from jax.experimental import pallas as pl
from jax.experimental.pallas import tpu as pltpu

NDCG_W = 0.4
LISTMLE_W = 0.3
BINARY_W = 0.3
K = 10
MARGIN = 0.1
LN2 = 0.6931471805599453

N_REAL = 20480
NROW = 256
NPAD = NROW * 128
FILLER = 0x7F800001


def _mangle(u):
    m = u >> 31
    return u ^ (m & 0x7FFFFFFF)


def _roll(x, shift, axis):
    return jnp.roll(x, shift, axis=axis)


def _body(p_ref, r_ref, a_ref, l_ref, out_ref):
    A = a_ref[...]
    Lab = l_ref[...]

    merged = jnp.where(Lab == 0, A, A - MARGIN)
    u = lax.bitcast_convert_type(merged, jnp.int32)
    s = _mangle(u)
    keys160 = (s & -2) | jnp.where(Lab == 1, 1, 0)
    x = jnp.concatenate(
        [keys160, jnp.full((NROW - 160, 128), FILLER, jnp.int32)], axis=0)

    iota_l = lax.broadcasted_iota(jnp.int32, (NROW, 128), 1)
    iota_r = lax.broadcasted_iota(jnp.int32, (NROW, 128), 0)

    for stage in range(1, 16):
        k = 1 << stage
        if k < 128:
            up = (iota_l & k) == 0
        elif k < NPAD:
            up = (iota_r & (k >> 7)) == 0
        else:
            up = None
        j = k >> 1
        while j >= 1:
            if j < 128:
                lower = (iota_l & j) == 0
                pm = _roll(x, -j, 1)
                pp = _roll(x, j, 1)
            else:
                jr = j >> 7
                lower = (iota_r & jr) == 0
                pm = _roll(x, -jr, 0)
                pp = _roll(x, jr, 0)
            p = jnp.where(lower, pm, pp)
            want_max = jnp.logical_xor(up, lower) if up is not None else ~lower
            x = jnp.where(want_max, jnp.maximum(x, p), jnp.minimum(x, p))
            j >>= 1

    pos_tag = (x & 1) == 1
    sk = x & -2
    v = lax.bitcast_convert_type(_mangle(sk), jnp.float32)
    flat = iota_r * 128 + iota_l
    real = flat < N_REAL
    negm = (~pos_tag) & real

    cnt = jnp.where(negm, 1.0, 0.0)
    val = jnp.where(negm, v, 0.0)

    for sh in (1, 2, 4, 8, 16, 32, 64):
        lm = iota_l >= sh
        cnt = cnt + jnp.where(lm, _roll(cnt, sh, 1), 0.0)
        val = val + jnp.where(lm, _roll(val, sh, 1), 0.0)
    rt_c = cnt[:, 127:128]
    rt_v = val[:, 127:128]
    ic = rt_c
    iv = rt_v
    iota_rc = iota_r[:, 0:1]
    for sh in (1, 2, 4, 8, 16, 32, 64, 128):
        rm = iota_rc >= sh
        ic = ic + jnp.where(rm, _roll(ic, sh, 0), 0.0)
        iv = iv + jnp.where(rm, _roll(iv, sh, 0), 0.0)
    cnt = cnt + (ic - rt_c)
    val = val + (iv - rt_v)
    n_neg_s = ic[NROW - 1:NROW, :]
    s_tot = iv[NROW - 1:NROW, :]

    contrib = jnp.where(pos_tag & real,
                        (s_tot - val) - v * (n_neg_s - cnt), 0.0)
    pair_sum = jnp.sum(contrib)

    P = p_ref[...]
    R = r_ref[...]
    D, B = P.shape
    idx = lax.broadcasted_iota(jnp.int32, (D, B), 0)

    rank_p = jnp.zeros((D, B), jnp.float32)
    rank_r = jnp.zeros((D, B), jnp.float32)
    s_exp = jnp.zeros((D, B), jnp.float32)
    for j in range(D):
        Pj = P[j:j + 1, :]
        Rj = R[j:j + 1, :]
        beats_p = (Pj > P) | ((Pj == P) & (j < idx))
        beats_r = (Rj > R) | ((Rj == R) & (j < idx))
        rank_p = rank_p + beats_p.astype(jnp.float32)
        rank_r = rank_r + beats_r.astype(jnp.float32)
        s_exp = s_exp + jnp.exp(Pj) * (1.0 - beats_r.astype(jnp.float32))

    w_p = jnp.where(rank_p < K, LN2 / jnp.log(rank_p + 2.0), 0.0)
    w_r = jnp.where(rank_r < K, LN2 / jnp.log(rank_r + 2.0), 0.0)
    dcg = jnp.sum(R * w_p, axis=0, keepdims=True)
    idcg = jnp.sum(R * w_r, axis=0, keepdims=True)
    ndcg_loss = 1.0 - jnp.sum(dcg / (idcg + 1e-8)) / B

    listmle = jnp.sum(jnp.log(s_exp + 1e-8) - P) / B

    pos = (Lab == 1).astype(jnp.float32)
    n_pos = jnp.sum(pos)
    n_neg = jnp.float32(N_REAL) - n_pos
    bce = jnp.sum(jnp.maximum(A, 0.0) - A * pos
                  + jnp.log(1.0 + jnp.exp(-jnp.abs(A)))) / N_REAL
    rank_loss = pair_sum / jnp.maximum(n_pos * n_neg, 1.0)
    bin_loss = jnp.where((n_pos > 0) & (n_neg > 0), rank_loss, bce)

    total = NDCG_W * ndcg_loss + LISTMLE_W * listmle + BINARY_W * bin_loss
    out_ref[...] = jnp.reshape(total, (1, 1))


def kernel(predictions, relevance_scores, labels):
    B, D = predictions.shape
    A = predictions.reshape(160, 128)
    L = labels.reshape(160, 128)
    PT = predictions.T
    RT = relevance_scores.T

    out = pl.pallas_call(
        _body,
        in_specs=[
            pl.BlockSpec((D, B), lambda: (0, 0)),
            pl.BlockSpec((D, B), lambda: (0, 0)),
            pl.BlockSpec((160, 128), lambda: (0, 0)),
            pl.BlockSpec((160, 128), lambda: (0, 0)),
        ],
        out_specs=pl.BlockSpec((1, 1), lambda: (0, 0)),
        out_shape=jax.ShapeDtypeStruct((1, 1), jnp.float32),
    )(PT, RT, A, L)
    return out.reshape(())
